# baseline (speedup 1.0000x reference)
import jax
import jax.numpy as jnp
from jax import lax
from jax.experimental import pallas as pl
from jax.experimental.pallas import tpu as pltpu
from jax.experimental.pallas import tpu_sc as plsc

NQ = 23
DIM = 1 << NQ
HALF = DIM // 2
QTR = DIM // 4
NC, NS = 2, 16
NW = NC * NS
BLEND_W = QTR // NW
COPY_W = HALF // NW
CB = 8192
NCHUNK = BLEND_W // CB


def _sc_body(state_hbm, ub_hbm, out_hbm, ubv, a0, a1, o0, o1, csem, _sem):
    w = lax.axis_index("s") * NC + lax.axis_index("c")
    cp = pltpu.async_copy(
        state_hbm.at[pl.ds(w * COPY_W, COPY_W)],
        out_hbm.at[pl.ds(w * COPY_W, COPY_W)],
        csem,
    )
    pltpu.sync_copy(ub_hbm, ubv)
    u00 = ubv[0, :]
    u01 = ubv[1, :]
    u10 = ubv[2, :]
    u11 = ubv[3, :]
    q2base = HALF + w * BLEND_W
    q3base = HALF + QTR + w * BLEND_W
    for c in range(NCHUNK):
        off = c * CB
        pltpu.sync_copy(state_hbm.at[pl.ds(q2base + off, CB)], a0)
        pltpu.sync_copy(state_hbm.at[pl.ds(q3base + off, CB)], a1)

        def body(j, carry):
            s = pl.ds(j * 16, 16)
            x = a0[s]
            y = a1[s]
            o0[s] = u00 * x + u01 * y
            o1[s] = u10 * x + u11 * y
            return carry

        lax.fori_loop(0, CB // 16, body, 0)
        pltpu.sync_copy(o0, out_hbm.at[pl.ds(q2base + off, CB)])
        pltpu.sync_copy(o1, out_hbm.at[pl.ds(q3base + off, CB)])
    cp.wait()


def kernel(state, U):
    ub = jnp.broadcast_to(U.astype(jnp.float32).reshape(4, 1), (4, 16))
    f = pl.kernel(
        _sc_body,
        out_type=jax.ShapeDtypeStruct((DIM,), jnp.float32),
        mesh=plsc.VectorSubcoreMesh(core_axis_name="c", subcore_axis_name="s"),
        scratch_types=[
            pltpu.VMEM((4, 16), jnp.float32),
            pltpu.VMEM((CB,), jnp.float32),
            pltpu.VMEM((CB,), jnp.float32),
            pltpu.VMEM((CB,), jnp.float32),
            pltpu.VMEM((CB,), jnp.float32),
            pltpu.SemaphoreType.DMA,
            pltpu.SemaphoreType.DMA,
        ],
    )
    return f(state, ub)
